# Initial kernel scaffold; baseline (speedup 1.0000x reference)
import jax
import jax.numpy as jnp
from jax.experimental import pallas as pl


def kernel(x, edge_index, edge_attr, fp_w1, fp_b1, ln1_g, ln1_b, fp_w2, fp_b2, ln2_g, ln2_b, lin_w1, att_src1, att_dst1, bias1, lam1, skip_w, skip_b, lin_w2, att_src2, att_dst2, bias2, lam2, p_w1, p_b1, p_w2, p_b2):
    raise NotImplementedError("write your pallas kernel here")


# speedup vs baseline: 2.1567x; 2.1567x over previous
import functools

import jax
import jax.numpy as jnp
from jax import lax
from jax.experimental import pallas as pl
from jax.experimental.pallas import tpu as pltpu
from jax.experimental.pallas import tpu_sc as plsc

_NBLK = 2000
_EBLK = 8000
_NPAD = 50048


def _gelu(x):
    return 0.5 * x * (1.0 + lax.erf(x * 0.7071067811865476))


def _ln(x, g, b, eps=1e-5):
    mu = jnp.mean(x, axis=-1, keepdims=True)
    xc = x - mu
    var = jnp.mean(xc * xc, axis=-1, keepdims=True)
    return xc * jax.lax.rsqrt(var + eps) * g + b


def _front_body(x_ref, w1_ref, b1_ref, g1_ref, be1_ref, w2_ref, b2_ref,
                g2_ref, be2_ref, wa_ref, h_ref, ap_ref):
    t = x_ref[...] @ w1_ref[...] + b1_ref[...]
    t = _gelu(_ln(t, g1_ref[...], be1_ref[...]))
    t = t @ w2_ref[...] + b2_ref[...]
    h = _gelu(_ln(t, g2_ref[...], be2_ref[...]))
    h_ref[...] = h
    ap_ref[...] = h @ wa_ref[...]


def _front(x, fp_w1, fp_b1, ln1_g, ln1_b, fp_w2, fp_b2, ln2_g, ln2_b, wa):
    n = x.shape[0]
    grid = n // _NBLK
    row = lambda shp: pl.BlockSpec(shp, lambda i: (i, 0))
    full = lambda shp: pl.BlockSpec(shp, lambda i: (0, 0))
    return pl.pallas_call(
        _front_body,
        grid=(grid,),
        in_specs=[row((_NBLK, 128)), full((128, 64)), full((1, 64)),
                  full((1, 64)), full((1, 64)), full((64, 128)),
                  full((1, 128)), full((1, 128)), full((1, 128)),
                  full((128, 8))],
        out_specs=[row((_NBLK, 128)), row((_NBLK, 8))],
        out_shape=[jax.ShapeDtypeStruct((n, 128), jnp.float32),
                   jax.ShapeDtypeStruct((n, 8), jnp.float32)],
    )(x, fp_w1, fp_b1[None, :], ln1_g[None, :], ln1_b[None, :], fp_w2,
      fp_b2[None, :], ln2_g[None, :], ln2_b[None, :], wa)


def _lin_slices_body(h_ref, w_ref, o_ref):
    o_ref[...] = (h_ref[...] @ w_ref[...][0])[None]


def _lin_slices(h, w, s):
    n, k = h.shape
    grid = (n // _NBLK, s)
    w_sl = w.reshape(k, s, 32).transpose(1, 0, 2)
    return pl.pallas_call(
        _lin_slices_body,
        grid=grid,
        in_specs=[pl.BlockSpec((_NBLK, k), lambda i, j: (i, 0)),
                  pl.BlockSpec((1, k, 32), lambda i, j: (j, 0, 0))],
        out_specs=pl.BlockSpec((1, _NBLK, 32), lambda i, j: (j, i, 0)),
        out_shape=jax.ShapeDtypeStruct((s, n, 32), jnp.float32),
    )(h, w_sl)


def _alpha(gd, gs, pairs, w, lam):
    cols = []
    for (a, b) in pairs:
        t = gd[:, a] + gs[:, b]
        cols.append(jnp.where(t >= 0, t, 0.2 * t))
    t = jnp.stack(cols, axis=-1)
    return t + lam * jnp.log(w + 1e-6)


def _edge_max_body(pairs, gd_ref, gs_ref, w_ref, lam_ref, o_ref):
    al = _alpha(gd_ref[...], gs_ref[...], pairs, w_ref[...], lam_ref[0, 0])
    o_ref[...] = jnp.broadcast_to(jnp.max(al)[None, None, None], (1, 1, 128))


def _edge_exp_body(pairs, gd_ref, gs_ref, w_ref, lam_ref, m_ref, o_ref):
    al = _alpha(gd_ref[...], gs_ref[...], pairs, w_ref[...], lam_ref[0, 0])
    ex = jnp.exp(al - m_ref[0, 0])
    o_ref[...] = jnp.pad(ex, ((0, 0), (0, 4 - len(pairs))))


def _edge_ex(gd, gs, w, lam, pairs):
    e = gd.shape[0]
    grid = e // _EBLK
    erow = lambda shp: pl.BlockSpec(shp, lambda i: (i, 0))
    full = lambda shp: pl.BlockSpec(shp, lambda i: (0, 0))
    wcol = w.reshape(e, 1)
    bmax = pl.pallas_call(
        functools.partial(_edge_max_body, pairs),
        grid=(grid,),
        in_specs=[erow((_EBLK, 8)), erow((_EBLK, 8)), erow((_EBLK, 1)),
                  full((1, 1))],
        out_specs=pl.BlockSpec((1, 1, 128), lambda i: (i, 0, 0)),
        out_shape=jax.ShapeDtypeStruct((grid, 1, 128), jnp.float32),
    )(gd, gs, wcol, lam.reshape(1, 1))
    m = jnp.max(bmax).reshape(1, 1)
    ex = pl.pallas_call(
        functools.partial(_edge_exp_body, pairs),
        grid=(grid,),
        in_specs=[erow((_EBLK, 8)), erow((_EBLK, 8)), erow((_EBLK, 1)),
                  full((1, 1)), full((1, 1))],
        out_specs=erow((_EBLK, 4)),
        out_shape=jax.ShapeDtypeStruct((e, 4), jnp.float32),
    )(gd, gs, wcol, lam.reshape(1, 1), m)
    return ex


def _epi1_body(agg_ref, den_ref, h_ref, b1_ref, sw_ref, sb_ref, w2_ref,
               wa_ref, xh2_ref, ap2_ref):
    den = den_ref[...][0] + den_ref[...][1] + 1e-16
    agg = agg_ref[...]
    cols = [agg[s] / den[:, s // 4][:, None] for s in range(16)]
    gat = jnp.concatenate(cols, axis=-1) + b1_ref[...]
    gat = jnp.where(gat > 0, gat, jnp.exp(jnp.minimum(gat, 0.0)) - 1.0)
    h2 = gat + h_ref[...] @ sw_ref[...] + sb_ref[...]
    xh2 = h2 @ w2_ref[...]
    for s in range(4):
        xh2_ref[s] = xh2[:, 32 * s:32 * s + 32]
    ap2_ref[...] = xh2 @ wa_ref[...]


def _epilogue1(agg, den, h, bias1, skip_w, skip_b, lin_w2, wa2):
    n = h.shape[0]
    grid = n // _NBLK
    full = lambda shp: pl.BlockSpec(shp, lambda i: tuple(0 for _ in shp))
    return pl.pallas_call(
        _epi1_body,
        grid=(grid,),
        in_specs=[pl.BlockSpec((16, _NBLK, 32), lambda i: (0, i, 0)),
                  pl.BlockSpec((2, _NBLK, 4), lambda i: (0, i, 0)),
                  pl.BlockSpec((_NBLK, 128), lambda i: (i, 0)),
                  full((1, 512)), full((128, 512)), full((1, 512)),
                  full((512, 128)), full((128, 8))],
        out_specs=[pl.BlockSpec((4, _NBLK, 32), lambda i: (0, i, 0)),
                   pl.BlockSpec((_NBLK, 8), lambda i: (i, 0))],
        out_shape=[jax.ShapeDtypeStruct((4, n, 32), jnp.float32),
                   jax.ShapeDtypeStruct((n, 8), jnp.float32)],
    )(agg, den, h, bias1[None, :], skip_w, skip_b[None, :], lin_w2, wa2)


def _epi2_body(agg_ref, den_ref, b2_ref, pw1_ref, pb1_ref, pw2_ref, pb2_ref,
               z_ref, emb_ref):
    den = den_ref[...][0] + den_ref[...][1] + 1e-16
    agg = agg_ref[...]
    cols = [agg[s] / den[:, 0][:, None] for s in range(4)]
    emb = jnp.concatenate(cols, axis=-1) + b2_ref[...]
    emb_ref[...] = emb
    t = jnp.maximum(emb @ pw1_ref[...] + pb1_ref[...], 0.0)
    z_ref[...] = t @ pw2_ref[...] + pb2_ref[...]


def _epilogue2(agg, den, bias2, p_w1, p_b1, p_w2, p_b2, n):
    grid = n // _NBLK
    full = lambda shp: pl.BlockSpec(shp, lambda i: tuple(0 for _ in shp))
    return pl.pallas_call(
        _epi2_body,
        grid=(grid,),
        in_specs=[pl.BlockSpec((4, _NBLK, 32), lambda i: (0, i, 0)),
                  pl.BlockSpec((2, _NBLK, 4), lambda i: (0, i, 0)),
                  full((1, 128)), full((128, 128)), full((1, 128)),
                  full((128, 32)), full((1, 32))],
        out_specs=[pl.BlockSpec((_NBLK, 32), lambda i: (i, 0)),
                   pl.BlockSpec((_NBLK, 128), lambda i: (i, 0))],
        out_shape=[jax.ShapeDtypeStruct((n, 32), jnp.float32),
                   jax.ShapeDtypeStruct((n, 128), jnp.float32)],
    )(agg, den, bias2[None, :], p_w1, p_b1[None, :], p_w2, p_b2[None, :])


def _edge_gather(ap, src, dst):
    return ap[dst], ap[src]


def _denom_scatter(ex, dst, n):
    d = jax.ops.segment_sum(ex, dst, num_segments=n)
    d = jnp.pad(d, ((0, _NPAD - n), (0, 0)))
    return jnp.stack([d, jnp.zeros_like(d)])


def _aggregate(table, src, dst, ex, s, n):
    heads = 4 if s == 16 else 1
    out = []
    for si in range(s):
        h = si // 4 if heads == 4 else 0
        msg = table[si][src] * ex[:, h][:, None]
        acc = jax.ops.segment_sum(msg, dst, num_segments=n)
        out.append(jnp.pad(acc, ((0, _NPAD - n), (0, 0))))
    return jnp.stack(out)


def kernel(x, edge_index, edge_attr, fp_w1, fp_b1, ln1_g, ln1_b, fp_w2,
           fp_b2, ln2_g, ln2_b, lin_w1, att_src1, att_dst1, bias1, lam1,
           skip_w, skip_b, lin_w2, att_src2, att_dst2, bias2, lam2, p_w1,
           p_b1, p_w2, p_b2):
    n = x.shape[0]
    src = edge_index[0]
    dst = edge_index[1]
    w = edge_attr

    h1 = att_src1.shape[1]
    af1 = jnp.zeros((h1 * 128, 8), jnp.float32)
    for h in range(h1):
        af1 = af1.at[h * 128:(h + 1) * 128, h].set(att_src1[0, h])
        af1 = af1.at[h * 128:(h + 1) * 128, 4 + h].set(att_dst1[0, h])
    wa1 = lin_w1 @ af1
    af2 = jnp.zeros((128, 8), jnp.float32)
    af2 = af2.at[:, 0].set(att_src2[0, 0])
    af2 = af2.at[:, 1].set(att_dst2[0, 0])
    wa2 = af2

    h, ap1 = _front(x, fp_w1, fp_b1, ln1_g, ln1_b, fp_w2, fp_b2, ln2_g,
                    ln2_b, wa1)
    xh1s = _lin_slices(h, lin_w1, 16)

    g1d, g1s = _edge_gather(ap1, src, dst)
    ex1 = _edge_ex(g1d, g1s, w, lam1, [(0, 4), (1, 5), (2, 6), (3, 7)])
    den1 = _denom_scatter(ex1, dst, n)
    agg1 = _aggregate(xh1s, src, dst, ex1, 16, n)

    xh2s, ap2 = _epilogue1(agg1[:, :n], den1[:, :n], h, bias1, skip_w,
                           skip_b, lin_w2, wa2)

    g2d, g2s = _edge_gather(ap2, src, dst)
    ex2 = _edge_ex(g2d, g2s, w, lam2, [(0, 1)])
    den2 = _denom_scatter(ex2, dst, n)
    agg2 = _aggregate(xh2s, src, dst, ex2, 4, n)

    z, emb = _epilogue2(agg2[:, :n], den2[:, :n], bias2, p_w1, p_b1, p_w2,
                        p_b2, n)
    return (z, emb)


# speedup vs baseline: 8.4022x; 3.8959x over previous
import functools

import jax
import jax.numpy as jnp
from jax import lax
from jax.experimental import pallas as pl
from jax.experimental.pallas import tpu as pltpu
from jax.experimental.pallas import tpu_sc as plsc

_NBLK = 2000
_EBLK = 6400
_NPAD = 50048


def _gelu(x):
    return 0.5 * x * (1.0 + lax.erf(x * 0.7071067811865476))


def _ln(x, g, b, eps=1e-5):
    mu = jnp.mean(x, axis=-1, keepdims=True)
    xc = x - mu
    var = jnp.mean(xc * xc, axis=-1, keepdims=True)
    return xc * jax.lax.rsqrt(var + eps) * g + b


def _front_body(x_ref, w1_ref, b1_ref, g1_ref, be1_ref, w2_ref, b2_ref,
                g2_ref, be2_ref, wa_ref, h_ref, ap_ref):
    t = x_ref[...] @ w1_ref[...] + b1_ref[...]
    t = _gelu(_ln(t, g1_ref[...], be1_ref[...]))
    t = t @ w2_ref[...] + b2_ref[...]
    h = _gelu(_ln(t, g2_ref[...], be2_ref[...]))
    h_ref[...] = h
    ap_ref[...] = h @ wa_ref[...]


def _front(x, fp_w1, fp_b1, ln1_g, ln1_b, fp_w2, fp_b2, ln2_g, ln2_b, wa):
    n = x.shape[0]
    grid = n // _NBLK
    row = lambda shp: pl.BlockSpec(shp, lambda i: (i, 0))
    full = lambda shp: pl.BlockSpec(shp, lambda i: (0, 0))
    return pl.pallas_call(
        _front_body,
        grid=(grid,),
        in_specs=[row((_NBLK, 128)), full((128, 64)), full((1, 64)),
                  full((1, 64)), full((1, 64)), full((64, 128)),
                  full((1, 128)), full((1, 128)), full((1, 128)),
                  full((128, 8))],
        out_specs=[row((_NBLK, 128)), row((_NBLK, 8))],
        out_shape=[jax.ShapeDtypeStruct((n, 128), jnp.float32),
                   jax.ShapeDtypeStruct((n, 8), jnp.float32)],
    )(x, fp_w1, fp_b1[None, :], ln1_g[None, :], ln1_b[None, :], fp_w2,
      fp_b2[None, :], ln2_g[None, :], ln2_b[None, :], wa)


def _lin_slices_body(h_ref, w_ref, o_ref):
    o_ref[...] = (h_ref[...] @ w_ref[...][0])[None]


def _lin_slices(h, w, s):
    n, k = h.shape
    grid = (n // _NBLK, s)
    w_sl = w.reshape(k, s, 16).transpose(1, 0, 2)
    return pl.pallas_call(
        _lin_slices_body,
        grid=grid,
        in_specs=[pl.BlockSpec((_NBLK, k), lambda i, j: (i, 0)),
                  pl.BlockSpec((1, k, 16), lambda i, j: (j, 0, 0))],
        out_specs=pl.BlockSpec((1, _NBLK, 16), lambda i, j: (j, i, 0)),
        out_shape=jax.ShapeDtypeStruct((s, n, 16), jnp.float32),
    )(h, w_sl)


def _alpha(gd, gs, pairs, w, lam):
    cols = []
    for (a, b) in pairs:
        t = gd[:, a] + gs[:, b]
        cols.append(jnp.where(t >= 0, t, 0.2 * t))
    t = jnp.stack(cols, axis=-1)
    return t + lam * jnp.log(w + 1e-6)


def _edge_max_body(pairs, gd_ref, gs_ref, w_ref, lam_ref, o_ref):
    al = _alpha(gd_ref[...], gs_ref[...], pairs, w_ref[...], lam_ref[0, 0])
    o_ref[...] = jnp.broadcast_to(jnp.max(al)[None, None, None], (1, 1, 128))


def _edge_exp_body(pairs, gd_ref, gs_ref, w_ref, lam_ref, m_ref, o_ref,
                   ot_ref):
    al = _alpha(gd_ref[...], gs_ref[...], pairs, w_ref[...], lam_ref[0, 0])
    ex = jnp.exp(al - m_ref[0, 0])
    o_ref[...] = jnp.pad(ex, ((0, 0), (0, 16 - len(pairs))))
    exp4 = jnp.pad(ex, ((0, 0), (0, 4 - len(pairs))))
    ot_ref[...] = exp4.T


def _edge_ex(gd, gs, w, lam, pairs):
    e = gd.shape[0]
    grid = e // _EBLK
    erow = lambda shp: pl.BlockSpec(shp, lambda i: (i, 0))
    full = lambda shp: pl.BlockSpec(shp, lambda i: (0, 0))
    wcol = w.reshape(e, 1)
    bmax = pl.pallas_call(
        functools.partial(_edge_max_body, pairs),
        grid=(grid,),
        in_specs=[erow((_EBLK, 8)), erow((_EBLK, 8)), erow((_EBLK, 1)),
                  full((1, 1))],
        out_specs=pl.BlockSpec((1, 1, 128), lambda i: (i, 0, 0)),
        out_shape=jax.ShapeDtypeStruct((grid, 1, 128), jnp.float32),
    )(gd, gs, wcol, lam.reshape(1, 1))
    m = jnp.max(bmax).reshape(1, 1)
    ex32, ex_t = pl.pallas_call(
        functools.partial(_edge_exp_body, pairs),
        grid=(grid,),
        in_specs=[erow((_EBLK, 8)), erow((_EBLK, 8)), erow((_EBLK, 1)),
                  full((1, 1)), full((1, 1))],
        out_specs=[erow((_EBLK, 16)),
                   pl.BlockSpec((4, _EBLK), lambda i: (0, i))],
        out_shape=[jax.ShapeDtypeStruct((e, 16), jnp.float32),
                   jax.ShapeDtypeStruct((4, e), jnp.float32)],
    )(gd, gs, wcol, lam.reshape(1, 1), m)
    return ex32, ex_t


def _epi1_body(agg_ref, den_ref, h_ref, b1_ref, sw_ref, sb_ref, w2_ref,
               wa_ref, xh2_ref, ap2_ref):
    den = den_ref[...][0] + den_ref[...][1] + 1e-16
    agg = agg_ref[...]
    cols = [agg[s] / den[:, s // 8][:, None] for s in range(32)]
    gat = jnp.concatenate(cols, axis=-1) + b1_ref[...]
    gat = jnp.where(gat > 0, gat, jnp.exp(jnp.minimum(gat, 0.0)) - 1.0)
    h2 = gat + h_ref[...] @ sw_ref[...] + sb_ref[...]
    xh2 = h2 @ w2_ref[...]
    for s in range(8):
        xh2_ref[s] = xh2[:, 16 * s:16 * s + 16]
    ap2_ref[...] = xh2 @ wa_ref[...]


def _epilogue1(agg, den, h, bias1, skip_w, skip_b, lin_w2, wa2):
    n = h.shape[0]
    blk = 400
    grid = n // blk
    full = lambda shp: pl.BlockSpec(shp, lambda i: tuple(0 for _ in shp))
    return pl.pallas_call(
        _epi1_body,
        grid=(grid,),
        in_specs=[pl.BlockSpec((32, blk, 16), lambda i: (0, i, 0)),
                  pl.BlockSpec((2, blk, 16), lambda i: (0, i, 0)),
                  pl.BlockSpec((blk, 128), lambda i: (i, 0)),
                  full((1, 512)), full((128, 512)), full((1, 512)),
                  full((512, 128)), full((128, 8))],
        out_specs=[pl.BlockSpec((8, blk, 16), lambda i: (0, i, 0)),
                   pl.BlockSpec((blk, 8), lambda i: (i, 0))],
        out_shape=[jax.ShapeDtypeStruct((8, n, 16), jnp.float32),
                   jax.ShapeDtypeStruct((n, 8), jnp.float32)],
    )(agg, den, h, bias1[None, :], skip_w, skip_b[None, :], lin_w2, wa2)


def _epi2_body(agg_ref, den_ref, b2_ref, pw1_ref, pb1_ref, pw2_ref, pb2_ref,
               z_ref, emb_ref):
    den = den_ref[...][0] + den_ref[...][1] + 1e-16
    agg = agg_ref[...]
    cols = [agg[s] / den[:, 0][:, None] for s in range(8)]
    emb = jnp.concatenate(cols, axis=-1) + b2_ref[...]
    emb_ref[...] = emb
    t = jnp.maximum(emb @ pw1_ref[...] + pb1_ref[...], 0.0)
    z_ref[...] = t @ pw2_ref[...] + pb2_ref[...]


def _epilogue2(agg, den, bias2, p_w1, p_b1, p_w2, p_b2, n):
    grid = n // _NBLK
    full = lambda shp: pl.BlockSpec(shp, lambda i: tuple(0 for _ in shp))
    return pl.pallas_call(
        _epi2_body,
        grid=(grid,),
        in_specs=[pl.BlockSpec((8, _NBLK, 16), lambda i: (0, i, 0)),
                  pl.BlockSpec((2, _NBLK, 16), lambda i: (0, i, 0)),
                  full((1, 128)), full((128, 128)), full((1, 128)),
                  full((128, 32)), full((1, 32))],
        out_specs=[pl.BlockSpec((_NBLK, 32), lambda i: (i, 0)),
                   pl.BlockSpec((_NBLK, 128), lambda i: (i, 0))],
        out_shape=[jax.ShapeDtypeStruct((n, 32), jnp.float32),
                   jax.ShapeDtypeStruct((n, 128), jnp.float32)],
    )(agg, den, bias2[None, :], p_w1, p_b1[None, :], p_w2, p_b2[None, :])


_NC, _NS = 2, 16
_NW = _NC * _NS


def _sc_mesh():
    return plsc.VectorSubcoreMesh(core_axis_name="c", subcore_axis_name="s")


def _sc_gather(ap, src, dst):
    e = src.shape[0]
    ew = e // _NW
    c = 5000
    nit = ew // c

    @functools.partial(
        pl.kernel,
        out_type=[jax.ShapeDtypeStruct((e, 8), jnp.float32),
                  jax.ShapeDtypeStruct((e, 8), jnp.float32)],
        mesh=_sc_mesh(),
        scratch_types=[pltpu.VMEM((c,), jnp.int32),
                       pltpu.VMEM((c, 8), jnp.float32),
                       pltpu.SemaphoreType.DMA],
        compiler_params=pltpu.CompilerParams(use_tc_tiling_on_sc=False),
    )
    def k(ap_h, src_h, dst_h, gd_h, gs_h, idx_v, rows_v, sem):
        wid = lax.axis_index("s") * _NC + lax.axis_index("c")
        for j in range(nit):
            base = pl.multiple_of(wid * ew + j * c, 8)
            pltpu.sync_copy(dst_h.at[pl.ds(base, c)], idx_v)
            pltpu.async_copy(ap_h.at[idx_v], rows_v, sem).wait()
            pltpu.sync_copy(rows_v, gd_h.at[pl.ds(base, c)])
            pltpu.sync_copy(src_h.at[pl.ds(base, c)], idx_v)
            pltpu.async_copy(ap_h.at[idx_v], rows_v, sem).wait()
            pltpu.sync_copy(rows_v, gs_h.at[pl.ds(base, c)])

    return k(ap, src, dst)


def _sc_denom(ex, dst, zpad):
    e = dst.shape[0]
    eh = e // _NC
    ew = eh // _NS
    stripe = _NPAD // _NS
    nsub, rem = divmod(ew, 128)

    @functools.partial(
        pl.kernel,
        out_type=jax.ShapeDtypeStruct((_NC * _NPAD, 16), jnp.float32),
        mesh=_sc_mesh(),
        scratch_types=[pltpu.VMEM((stripe, 16), jnp.float32),
                       pltpu.VMEM((128,), jnp.int32),
                       pltpu.VMEM((128, 16), jnp.float32),
                       pltpu.VMEM((max(rem, 8), 16), jnp.float32),
                       pltpu.VMEM((max(rem, 8),), jnp.int32),
                       pltpu.VMEM_SHARED((_NPAD, 16), jnp.float32),
                       pltpu.SemaphoreType.DMA],
        compiler_params=pltpu.CompilerParams(use_tc_tiling_on_sc=False),
    )
    def k(ex_h, dst_h, z_h, den_h, buf_v, i128, r128, r8, i8, slab, sem):
        cid = lax.axis_index("c")
        sid = lax.axis_index("s")
        so = pl.multiple_of(sid * stripe, 8)
        pltpu.sync_copy(z_h.at[pl.ds(so, stripe)], buf_v)
        pltpu.sync_copy(buf_v, slab.at[pl.ds(so, stripe)])
        plsc.subcore_barrier()
        tbase = pl.multiple_of(cid * eh + sid * ew, 8)

        def sub(k2, _):
            o = pl.multiple_of(tbase + k2 * 128, 8)
            pltpu.sync_copy(dst_h.at[pl.ds(o, 128)], i128)
            pltpu.sync_copy(ex_h.at[pl.ds(o, 128)], r128)
            pltpu.sync_copy(r128, slab.at[i128], add=True)
            return _

        lax.fori_loop(0, nsub, sub, 0)
        if rem:
            o = pl.multiple_of(tbase + nsub * 128, 8)
            pltpu.sync_copy(dst_h.at[pl.ds(o, rem)], i8)
            pltpu.sync_copy(ex_h.at[pl.ds(o, rem)], r8)
            pltpu.sync_copy(r8, slab.at[i8], add=True)
        plsc.subcore_barrier()
        pltpu.sync_copy(slab.at[pl.ds(so, stripe)], buf_v)
        pltpu.sync_copy(buf_v, den_h.at[pl.ds(cid * _NPAD + so, stripe)])

    return k(ex, dst, zpad).reshape(_NC, _NPAD, 16)


def _sc_agg(table, src, dst, ex_t, zpad, s, n):
    e = src.shape[0]
    spc = s // _NC
    ew = e // _NS
    c = 2000
    nchunk = ew // c
    assert ew % c == 0
    stripe = _NPAD // _NS
    nsub, rem = divmod(c, 128)

    @functools.partial(
        pl.kernel,
        out_type=jax.ShapeDtypeStruct((s * _NPAD, 16), jnp.float32),
        mesh=_sc_mesh(),
        scratch_types=[pltpu.VMEM((c, 16), jnp.float32),
                       pltpu.VMEM((c,), jnp.int32),
                       pltpu.VMEM((c,), jnp.float32),
                       pltpu.VMEM((128,), jnp.int32),
                       pltpu.VMEM((max(rem, 8),), jnp.int32),
                       pltpu.VMEM_SHARED((_NPAD, 16), jnp.float32),
                       pltpu.SemaphoreType.DMA],
        compiler_params=pltpu.CompilerParams(use_tc_tiling_on_sc=False),
    )
    def k(tab_h, src_h, dst_h, ext_h, z_h, out_h, rows_v, idx_v, exb_v,
          d128, d80, slab, sem):
        cid = lax.axis_index("c")
        sid = lax.axis_index("s")
        so = pl.multiple_of(sid * stripe, 8)

        def slice_body(ks, _0):
            sl = cid * spc + ks
            hd = sl // 8 if s == 32 else 0
            off = sl * n
            pltpu.sync_copy(z_h.at[pl.ds(so, c)], rows_v)
            pltpu.sync_copy(rows_v, slab.at[pl.ds(so, c)])
            pltpu.sync_copy(z_h.at[pl.ds(so + c, stripe - c)],
                            rows_v.at[pl.ds(0, stripe - c)])
            pltpu.sync_copy(rows_v.at[pl.ds(0, stripe - c)],
                            slab.at[pl.ds(so + c, stripe - c)])
            plsc.subcore_barrier()

            def chunk(j, _):
                base = pl.multiple_of(sid * ew + j * c, 8)
                pltpu.sync_copy(src_h.at[pl.ds(base, c)], idx_v)
                for t in range(c // 16):
                    iv = idx_v[pl.ds(t * 16, 16)]
                    idx_v[pl.ds(t * 16, 16)] = iv + off
                pltpu.async_copy(tab_h.at[idx_v], rows_v, sem).wait()
                pltpu.sync_copy(ext_h.at[hd, pl.ds(base, c)], exb_v)

                def scale(g, _2):
                    exv = exb_v[pl.ds(g * 16, 16)]
                    for lane in range(16):
                        j2 = g * 16 + lane
                        ev = exv[lane]
                        rows_v[j2, pl.ds(0, 16)] = (
                            rows_v[j2, pl.ds(0, 16)] * ev)
                    return _2

                lax.fori_loop(0, c // 16, scale, 0)
                for q in range(nsub):
                    o = pl.multiple_of(q * 128, 8)
                    pltpu.sync_copy(dst_h.at[pl.ds(base + o, 128)], d128)
                    pltpu.sync_copy(rows_v.at[pl.ds(o, 128)],
                                    slab.at[d128], add=True)
                if rem:
                    o = pl.multiple_of(nsub * 128, 8)
                    pltpu.sync_copy(dst_h.at[pl.ds(base + o, rem)], d80)
                    pltpu.sync_copy(rows_v.at[pl.ds(o, rem)],
                                    slab.at[d80], add=True)
                return _

            lax.fori_loop(0, nchunk, chunk, 0)
            plsc.subcore_barrier()
            pltpu.sync_copy(slab.at[pl.ds(so, c)], rows_v)
            pltpu.sync_copy(rows_v, out_h.at[pl.ds(sl * _NPAD + so, c)])
            pltpu.sync_copy(slab.at[pl.ds(so + c, stripe - c)],
                            rows_v.at[pl.ds(0, stripe - c)])
            pltpu.sync_copy(rows_v.at[pl.ds(0, stripe - c)],
                            out_h.at[pl.ds(sl * _NPAD + so + c, stripe - c)])
            plsc.subcore_barrier()
            return _0

        lax.fori_loop(0, spc, slice_body, 0)

    return k(table, src, dst, ex_t, zpad).reshape(s, _NPAD, 16)


def _edge_gather(ap, src, dst):
    return _sc_gather(ap, src, dst)


def _denom_scatter(ex32, dst, n):
    del n
    zpad = jnp.zeros((_NPAD, 16), jnp.float32)
    return _sc_denom(ex32, dst, zpad)


def _aggregate(table, src, dst, ex_t, s, n):
    zpad = jnp.zeros((_NPAD, 16), jnp.float32)
    return _sc_agg(table.reshape(s * n, 16), src, dst, ex_t, zpad, s, n)


def kernel(x, edge_index, edge_attr, fp_w1, fp_b1, ln1_g, ln1_b, fp_w2,
           fp_b2, ln2_g, ln2_b, lin_w1, att_src1, att_dst1, bias1, lam1,
           skip_w, skip_b, lin_w2, att_src2, att_dst2, bias2, lam2, p_w1,
           p_b1, p_w2, p_b2):
    n = x.shape[0]
    src = edge_index[0]
    dst = edge_index[1]
    w = edge_attr

    h1 = att_src1.shape[1]
    af1 = jnp.zeros((h1 * 128, 8), jnp.float32)
    for h in range(h1):
        af1 = af1.at[h * 128:(h + 1) * 128, h].set(att_src1[0, h])
        af1 = af1.at[h * 128:(h + 1) * 128, 4 + h].set(att_dst1[0, h])
    wa1 = lin_w1 @ af1
    af2 = jnp.zeros((128, 8), jnp.float32)
    af2 = af2.at[:, 0].set(att_src2[0, 0])
    af2 = af2.at[:, 1].set(att_dst2[0, 0])
    wa2 = af2

    h, ap1 = _front(x, fp_w1, fp_b1, ln1_g, ln1_b, fp_w2, fp_b2, ln2_g,
                    ln2_b, wa1)
    xh1s = _lin_slices(h, lin_w1, 32)

    g1d, g1s = _edge_gather(ap1, src, dst)
    ex1, ex1t = _edge_ex(g1d, g1s, w, lam1, [(0, 4), (1, 5), (2, 6), (3, 7)])
    den1 = _denom_scatter(ex1, dst, n)
    agg1 = _aggregate(xh1s, src, dst, ex1t, 32, n)

    xh2s, ap2 = _epilogue1(agg1[:, :n], den1[:, :n], h, bias1, skip_w,
                           skip_b, lin_w2, wa2)

    g2d, g2s = _edge_gather(ap2, src, dst)
    ex2, ex2t = _edge_ex(g2d, g2s, w, lam2, [(0, 1)])
    den2 = _denom_scatter(ex2, dst, n)
    agg2 = _aggregate(xh2s, src, dst, ex2t, 8, n)

    z, emb = _epilogue2(agg2[:, :n], den2[:, :n], bias2, p_w1, p_b1, p_w2,
                        p_b2, n)
    return (z, emb)


# speedup vs baseline: 8.6252x; 1.0265x over previous
import functools

import jax
import jax.numpy as jnp
from jax import lax
from jax.experimental import pallas as pl
from jax.experimental.pallas import tpu as pltpu
from jax.experimental.pallas import tpu_sc as plsc

_NBLK = 2000
_EBLK = 6400
_NPAD = 50048


def _gelu(x):
    return 0.5 * x * (1.0 + lax.erf(x * 0.7071067811865476))


def _ln(x, g, b, eps=1e-5):
    mu = jnp.mean(x, axis=-1, keepdims=True)
    xc = x - mu
    var = jnp.mean(xc * xc, axis=-1, keepdims=True)
    return xc * jax.lax.rsqrt(var + eps) * g + b


def _front_body(x_ref, w1_ref, b1_ref, g1_ref, be1_ref, w2_ref, b2_ref,
                g2_ref, be2_ref, wa_ref, h_ref, ap_ref):
    t = x_ref[...] @ w1_ref[...] + b1_ref[...]
    t = _gelu(_ln(t, g1_ref[...], be1_ref[...]))
    t = t @ w2_ref[...] + b2_ref[...]
    h = _gelu(_ln(t, g2_ref[...], be2_ref[...]))
    h_ref[...] = h
    ap_ref[...] = h @ wa_ref[...]


def _front(x, fp_w1, fp_b1, ln1_g, ln1_b, fp_w2, fp_b2, ln2_g, ln2_b, wa):
    n = x.shape[0]
    grid = n // _NBLK
    row = lambda shp: pl.BlockSpec(shp, lambda i: (i, 0))
    full = lambda shp: pl.BlockSpec(shp, lambda i: (0, 0))
    return pl.pallas_call(
        _front_body,
        grid=(grid,),
        in_specs=[row((_NBLK, 128)), full((128, 64)), full((1, 64)),
                  full((1, 64)), full((1, 64)), full((64, 128)),
                  full((1, 128)), full((1, 128)), full((1, 128)),
                  full((128, 8))],
        out_specs=[row((_NBLK, 128)), row((_NBLK, 8))],
        out_shape=[jax.ShapeDtypeStruct((n, 128), jnp.float32),
                   jax.ShapeDtypeStruct((n, 8), jnp.float32)],
    )(x, fp_w1, fp_b1[None, :], ln1_g[None, :], ln1_b[None, :], fp_w2,
      fp_b2[None, :], ln2_g[None, :], ln2_b[None, :], wa)


def _lin_slices_body(s, h_ref, w_ref, o_ref):
    xh = h_ref[...] @ w_ref[...]
    for si in range(s):
        o_ref[si] = xh[:, 16 * si:16 * si + 16]


def _lin_slices(h, w, s):
    n, k = h.shape
    blk = 1000
    grid = n // blk
    return pl.pallas_call(
        functools.partial(_lin_slices_body, s),
        grid=(grid,),
        in_specs=[pl.BlockSpec((blk, k), lambda i: (i, 0)),
                  pl.BlockSpec((k, s * 16), lambda i: (0, 0))],
        out_specs=pl.BlockSpec((s, blk, 16), lambda i: (0, i, 0)),
        out_shape=jax.ShapeDtypeStruct((s, n, 16), jnp.float32),
    )(h, w)


def _alpha(gd, gs, pairs, w, lam):
    cols = []
    for (a, b) in pairs:
        t = gd[:, a] + gs[:, b]
        cols.append(jnp.where(t >= 0, t, 0.2 * t))
    t = jnp.stack(cols, axis=-1)
    return t + lam * jnp.log(w + 1e-6)


def _edge_max_body(pairs, gd_ref, gs_ref, w_ref, lam_ref, o_ref):
    al = _alpha(gd_ref[...], gs_ref[...], pairs, w_ref[...], lam_ref[0, 0])
    o_ref[...] = jnp.broadcast_to(jnp.max(al)[None, None, None], (1, 1, 128))


def _edge_exp_body(pairs, gd_ref, gs_ref, w_ref, lam_ref, m_ref, o_ref,
                   ot_ref):
    al = _alpha(gd_ref[...], gs_ref[...], pairs, w_ref[...], lam_ref[0, 0])
    ex = jnp.exp(al - m_ref[0, 0])
    o_ref[...] = jnp.pad(ex, ((0, 0), (0, 16 - len(pairs))))
    exp4 = jnp.pad(ex, ((0, 0), (0, 4 - len(pairs))))
    ot_ref[...] = exp4.T


def _edge_ex(gd, gs, w, lam, pairs):
    e = gd.shape[0]
    grid = e // _EBLK
    erow = lambda shp: pl.BlockSpec(shp, lambda i: (i, 0))
    full = lambda shp: pl.BlockSpec(shp, lambda i: (0, 0))
    wcol = w.reshape(e, 1)
    bmax = pl.pallas_call(
        functools.partial(_edge_max_body, pairs),
        grid=(grid,),
        in_specs=[erow((_EBLK, 8)), erow((_EBLK, 8)), erow((_EBLK, 1)),
                  full((1, 1))],
        out_specs=pl.BlockSpec((1, 1, 128), lambda i: (i, 0, 0)),
        out_shape=jax.ShapeDtypeStruct((grid, 1, 128), jnp.float32),
    )(gd, gs, wcol, lam.reshape(1, 1))
    m = jnp.max(bmax).reshape(1, 1)
    ex32, ex_t = pl.pallas_call(
        functools.partial(_edge_exp_body, pairs),
        grid=(grid,),
        in_specs=[erow((_EBLK, 8)), erow((_EBLK, 8)), erow((_EBLK, 1)),
                  full((1, 1)), full((1, 1))],
        out_specs=[erow((_EBLK, 16)),
                   pl.BlockSpec((4, _EBLK), lambda i: (0, i))],
        out_shape=[jax.ShapeDtypeStruct((e, 16), jnp.float32),
                   jax.ShapeDtypeStruct((4, e), jnp.float32)],
    )(gd, gs, wcol, lam.reshape(1, 1), m)
    return ex32, ex_t


def _epi1_body(agg_ref, den_ref, h_ref, b1_ref, sw_ref, sb_ref, w2_ref,
               wa_ref, xh2_ref, ap2_ref):
    den = den_ref[...][0] + den_ref[...][1] + 1e-16
    agg = agg_ref[...]
    cols = [agg[s] / den[:, s // 8][:, None] for s in range(32)]
    gat = jnp.concatenate(cols, axis=-1) + b1_ref[...]
    gat = jnp.where(gat > 0, gat, jnp.exp(jnp.minimum(gat, 0.0)) - 1.0)
    h2 = gat + h_ref[...] @ sw_ref[...] + sb_ref[...]
    xh2 = h2 @ w2_ref[...]
    for s in range(8):
        xh2_ref[s] = xh2[:, 16 * s:16 * s + 16]
    ap2_ref[...] = xh2 @ wa_ref[...]


def _epilogue1(agg, den, h, bias1, skip_w, skip_b, lin_w2, wa2):
    n = h.shape[0]
    blk = 400
    grid = n // blk
    full = lambda shp: pl.BlockSpec(shp, lambda i: tuple(0 for _ in shp))
    return pl.pallas_call(
        _epi1_body,
        grid=(grid,),
        in_specs=[pl.BlockSpec((32, blk, 16), lambda i: (0, i, 0)),
                  pl.BlockSpec((2, blk, 16), lambda i: (0, i, 0)),
                  pl.BlockSpec((blk, 128), lambda i: (i, 0)),
                  full((1, 512)), full((128, 512)), full((1, 512)),
                  full((512, 128)), full((128, 8))],
        out_specs=[pl.BlockSpec((8, blk, 16), lambda i: (0, i, 0)),
                   pl.BlockSpec((blk, 8), lambda i: (i, 0))],
        out_shape=[jax.ShapeDtypeStruct((8, n, 16), jnp.float32),
                   jax.ShapeDtypeStruct((n, 8), jnp.float32)],
    )(agg, den, h, bias1[None, :], skip_w, skip_b[None, :], lin_w2, wa2)


def _epi2_body(agg_ref, den_ref, b2_ref, pw1_ref, pb1_ref, pw2_ref, pb2_ref,
               z_ref, emb_ref):
    den = den_ref[...][0] + den_ref[...][1] + 1e-16
    agg = agg_ref[...]
    cols = [agg[s] / den[:, 0][:, None] for s in range(8)]
    emb = jnp.concatenate(cols, axis=-1) + b2_ref[...]
    emb_ref[...] = emb
    t = jnp.maximum(emb @ pw1_ref[...] + pb1_ref[...], 0.0)
    z_ref[...] = t @ pw2_ref[...] + pb2_ref[...]


def _epilogue2(agg, den, bias2, p_w1, p_b1, p_w2, p_b2, n):
    grid = n // _NBLK
    full = lambda shp: pl.BlockSpec(shp, lambda i: tuple(0 for _ in shp))
    return pl.pallas_call(
        _epi2_body,
        grid=(grid,),
        in_specs=[pl.BlockSpec((8, _NBLK, 16), lambda i: (0, i, 0)),
                  pl.BlockSpec((2, _NBLK, 16), lambda i: (0, i, 0)),
                  full((1, 128)), full((128, 128)), full((1, 128)),
                  full((128, 32)), full((1, 32))],
        out_specs=[pl.BlockSpec((_NBLK, 32), lambda i: (i, 0)),
                   pl.BlockSpec((_NBLK, 128), lambda i: (i, 0))],
        out_shape=[jax.ShapeDtypeStruct((n, 32), jnp.float32),
                   jax.ShapeDtypeStruct((n, 128), jnp.float32)],
    )(agg, den, bias2[None, :], p_w1, p_b1[None, :], p_w2, p_b2[None, :])


_NC, _NS = 2, 16
_NW = _NC * _NS


def _sc_mesh():
    return plsc.VectorSubcoreMesh(core_axis_name="c", subcore_axis_name="s")


def _sc_gather(ap, src, dst):
    e = src.shape[0]
    ew = e // _NW
    c = 5000
    nit = ew // c

    @functools.partial(
        pl.kernel,
        out_type=[jax.ShapeDtypeStruct((e, 8), jnp.float32),
                  jax.ShapeDtypeStruct((e, 8), jnp.float32)],
        mesh=_sc_mesh(),
        scratch_types=[pltpu.VMEM((c,), jnp.int32),
                       pltpu.VMEM((c, 8), jnp.float32),
                       pltpu.SemaphoreType.DMA],
        compiler_params=pltpu.CompilerParams(use_tc_tiling_on_sc=False),
    )
    def k(ap_h, src_h, dst_h, gd_h, gs_h, idx_v, rows_v, sem):
        wid = lax.axis_index("s") * _NC + lax.axis_index("c")
        for j in range(nit):
            base = pl.multiple_of(wid * ew + j * c, 8)
            pltpu.sync_copy(dst_h.at[pl.ds(base, c)], idx_v)
            pltpu.async_copy(ap_h.at[idx_v], rows_v, sem).wait()
            pltpu.sync_copy(rows_v, gd_h.at[pl.ds(base, c)])
            pltpu.sync_copy(src_h.at[pl.ds(base, c)], idx_v)
            pltpu.async_copy(ap_h.at[idx_v], rows_v, sem).wait()
            pltpu.sync_copy(rows_v, gs_h.at[pl.ds(base, c)])

    return k(ap, src, dst)


def _sc_denom(ex, dst, zpad):
    e = dst.shape[0]
    eh = e // _NC
    ew = eh // _NS
    stripe = _NPAD // _NS
    nsub, rem = divmod(ew, 128)

    @functools.partial(
        pl.kernel,
        out_type=jax.ShapeDtypeStruct((_NC * _NPAD, 16), jnp.float32),
        mesh=_sc_mesh(),
        scratch_types=[pltpu.VMEM((stripe, 16), jnp.float32),
                       pltpu.VMEM((128,), jnp.int32),
                       pltpu.VMEM((128, 16), jnp.float32),
                       pltpu.VMEM((max(rem, 8), 16), jnp.float32),
                       pltpu.VMEM((max(rem, 8),), jnp.int32),
                       pltpu.VMEM_SHARED((_NPAD, 16), jnp.float32),
                       pltpu.SemaphoreType.DMA],
        compiler_params=pltpu.CompilerParams(use_tc_tiling_on_sc=False),
    )
    def k(ex_h, dst_h, z_h, den_h, buf_v, i128, r128, r8, i8, slab, sem):
        cid = lax.axis_index("c")
        sid = lax.axis_index("s")
        so = pl.multiple_of(sid * stripe, 8)
        pltpu.sync_copy(z_h.at[pl.ds(so, stripe)], buf_v)
        pltpu.sync_copy(buf_v, slab.at[pl.ds(so, stripe)])
        plsc.subcore_barrier()
        tbase = pl.multiple_of(cid * eh + sid * ew, 8)

        def sub(k2, _):
            o = pl.multiple_of(tbase + k2 * 128, 8)
            pltpu.sync_copy(dst_h.at[pl.ds(o, 128)], i128)
            pltpu.sync_copy(ex_h.at[pl.ds(o, 128)], r128)
            pltpu.sync_copy(r128, slab.at[i128], add=True)
            return _

        lax.fori_loop(0, nsub, sub, 0)
        if rem:
            o = pl.multiple_of(tbase + nsub * 128, 8)
            pltpu.sync_copy(dst_h.at[pl.ds(o, rem)], i8)
            pltpu.sync_copy(ex_h.at[pl.ds(o, rem)], r8)
            pltpu.sync_copy(r8, slab.at[i8], add=True)
        plsc.subcore_barrier()
        pltpu.sync_copy(slab.at[pl.ds(so, stripe)], buf_v)
        pltpu.sync_copy(buf_v, den_h.at[pl.ds(cid * _NPAD + so, stripe)])

    return k(ex, dst, zpad).reshape(_NC, _NPAD, 16)


def _sc_agg(table, src, dst, ex_t, zpad, s, n):
    e = src.shape[0]
    spc = s // _NC
    ew = e // _NS
    c = 2000
    nchunk = ew // c
    assert ew % c == 0
    stripe = _NPAD // _NS
    nsub, rem = divmod(c, 128)

    @functools.partial(
        pl.kernel,
        out_type=jax.ShapeDtypeStruct((s * _NPAD, 16), jnp.float32),
        mesh=_sc_mesh(),
        scratch_types=[pltpu.VMEM((c, 16), jnp.float32),
                       pltpu.VMEM((c,), jnp.int32),
                       pltpu.VMEM((c,), jnp.float32),
                       pltpu.VMEM((128,), jnp.int32),
                       pltpu.VMEM((max(rem, 8),), jnp.int32),
                       pltpu.VMEM_SHARED((_NPAD, 16), jnp.float32),
                       pltpu.SemaphoreType.DMA],
        compiler_params=pltpu.CompilerParams(use_tc_tiling_on_sc=False),
    )
    def k(tab_h, src_h, dst_h, ext_h, z_h, out_h, rows_v, idx_v, exb_v,
          d128, d80, slab, sem):
        cid = lax.axis_index("c")
        sid = lax.axis_index("s")
        so = pl.multiple_of(sid * stripe, 8)

        def slice_body(ks, _0):
            sl = cid * spc + ks
            hd = sl // 8 if s == 32 else 0
            off = sl * n
            pltpu.sync_copy(z_h.at[pl.ds(so, c)], rows_v)
            pltpu.sync_copy(rows_v, slab.at[pl.ds(so, c)])
            pltpu.sync_copy(z_h.at[pl.ds(so + c, stripe - c)],
                            rows_v.at[pl.ds(0, stripe - c)])
            pltpu.sync_copy(rows_v.at[pl.ds(0, stripe - c)],
                            slab.at[pl.ds(so + c, stripe - c)])
            plsc.subcore_barrier()

            def chunk(j, _):
                base = pl.multiple_of(sid * ew + j * c, 8)
                pltpu.sync_copy(src_h.at[pl.ds(base, c)], idx_v)
                for t in range(c // 16):
                    iv = idx_v[pl.ds(t * 16, 16)]
                    idx_v[pl.ds(t * 16, 16)] = iv + off
                pltpu.async_copy(tab_h.at[idx_v], rows_v, sem).wait()
                pltpu.sync_copy(ext_h.at[hd, pl.ds(base, c)], exb_v)

                def scale(g, _2):
                    exv = exb_v[pl.ds(g * 16, 16)]
                    for lane in range(16):
                        j2 = g * 16 + lane
                        ev = exv[lane]
                        rows_v[j2, pl.ds(0, 16)] = (
                            rows_v[j2, pl.ds(0, 16)] * ev)
                    return _2

                lax.fori_loop(0, c // 16, scale, 0)
                for q in range(nsub):
                    o = pl.multiple_of(q * 128, 8)
                    pltpu.sync_copy(dst_h.at[pl.ds(base + o, 128)], d128)
                    pltpu.sync_copy(rows_v.at[pl.ds(o, 128)],
                                    slab.at[d128], add=True)
                if rem:
                    o = pl.multiple_of(nsub * 128, 8)
                    pltpu.sync_copy(dst_h.at[pl.ds(base + o, rem)], d80)
                    pltpu.sync_copy(rows_v.at[pl.ds(o, rem)],
                                    slab.at[d80], add=True)
                return _

            lax.fori_loop(0, nchunk, chunk, 0)
            plsc.subcore_barrier()
            pltpu.sync_copy(slab.at[pl.ds(so, c)], rows_v)
            pltpu.sync_copy(rows_v, out_h.at[pl.ds(sl * _NPAD + so, c)])
            pltpu.sync_copy(slab.at[pl.ds(so + c, stripe - c)],
                            rows_v.at[pl.ds(0, stripe - c)])
            pltpu.sync_copy(rows_v.at[pl.ds(0, stripe - c)],
                            out_h.at[pl.ds(sl * _NPAD + so + c, stripe - c)])
            plsc.subcore_barrier()
            return _0

        lax.fori_loop(0, spc, slice_body, 0)

    return k(table, src, dst, ex_t, zpad).reshape(s, _NPAD, 16)


def _edge_gather(ap, src, dst):
    return _sc_gather(ap, src, dst)


def _denom_scatter(ex32, dst, n):
    del n
    zpad = jnp.zeros((_NPAD, 16), jnp.float32)
    return _sc_denom(ex32, dst, zpad)


def _aggregate(table, src, dst, ex_t, s, n):
    zpad = jnp.zeros((_NPAD, 16), jnp.float32)
    return _sc_agg(table.reshape(s * n, 16), src, dst, ex_t, zpad, s, n)


def kernel(x, edge_index, edge_attr, fp_w1, fp_b1, ln1_g, ln1_b, fp_w2,
           fp_b2, ln2_g, ln2_b, lin_w1, att_src1, att_dst1, bias1, lam1,
           skip_w, skip_b, lin_w2, att_src2, att_dst2, bias2, lam2, p_w1,
           p_b1, p_w2, p_b2):
    n = x.shape[0]
    src = edge_index[0]
    dst = edge_index[1]
    w = edge_attr

    h1 = att_src1.shape[1]
    af1 = jnp.zeros((h1 * 128, 8), jnp.float32)
    for h in range(h1):
        af1 = af1.at[h * 128:(h + 1) * 128, h].set(att_src1[0, h])
        af1 = af1.at[h * 128:(h + 1) * 128, 4 + h].set(att_dst1[0, h])
    wa1 = lin_w1 @ af1
    af2 = jnp.zeros((128, 8), jnp.float32)
    af2 = af2.at[:, 0].set(att_src2[0, 0])
    af2 = af2.at[:, 1].set(att_dst2[0, 0])
    wa2 = af2

    h, ap1 = _front(x, fp_w1, fp_b1, ln1_g, ln1_b, fp_w2, fp_b2, ln2_g,
                    ln2_b, wa1)
    xh1s = _lin_slices(h, lin_w1, 32)

    g1d, g1s = _edge_gather(ap1, src, dst)
    ex1, ex1t = _edge_ex(g1d, g1s, w, lam1, [(0, 4), (1, 5), (2, 6), (3, 7)])
    den1 = _denom_scatter(ex1, dst, n)
    agg1 = _aggregate(xh1s, src, dst, ex1t, 32, n)

    xh2s, ap2 = _epilogue1(agg1[:, :n], den1[:, :n], h, bias1, skip_w,
                           skip_b, lin_w2, wa2)

    g2d, g2s = _edge_gather(ap2, src, dst)
    ex2, ex2t = _edge_ex(g2d, g2s, w, lam2, [(0, 1)])
    den2 = _denom_scatter(ex2, dst, n)
    agg2 = _aggregate(xh2s, src, dst, ex2t, 8, n)

    z, emb = _epilogue2(agg2[:, :n], den2[:, :n], bias2, p_w1, p_b1, p_w2,
                        p_b2, n)
    return (z, emb)
